# Initial kernel scaffold; baseline (speedup 1.0000x reference)
import jax
import jax.numpy as jnp
from jax.experimental import pallas as pl


def kernel(x, edge_index, batch, target, gin_w1, gin_b1, gin_w2, gin_b2, fcg_w, fcg_b, emb_table, conv_w, conv_b, fcxt_w, fcxt_b, fc1_w, fc1_b, fc2_w, fc2_b, out_w, out_b):
    raise NotImplementedError("write your pallas kernel here")


# speedup vs baseline: 2.9446x; 2.9446x over previous
import functools

import jax
import jax.numpy as jnp
from jax import lax
from jax.experimental import pallas as pl
from jax.experimental.pallas import tpu as pltpu
from jax.experimental.pallas import tpu_sc as plsc

N = 50000
E = 800000
G = 256
D = 78
H = 312
OUT = 128
EMB = 128
VOC = 26
KW = 32
CH = 4
SEQ = 1000
EPS = 1.583
XT_IN = CH * (EMB - KW + 1)

NPAD = 50176
HP = 320
DP = 80

NC = 2
NS = 16
HALF = 25000
ACC_ROWS = 25024
EDGES_PER_TILE = E // NS
CHUNK = 64
NFULL = EDGES_PER_TILE // CHUNK
TAIL = EDGES_PER_TILE - NFULL * CHUNK

ROWS_PER_TILE = NPAD // (NC * NS)
GM_ROWS = 272

_MESH = plsc.VectorSubcoreMesh(core_axis_name="c", subcore_axis_name="s")


def _segsum_body(x_hbm, src_hbm, dst_hbm, zeros_hbm, out_hbm,
                 sidx, dtmp, didx, rows, acc, gsem):
    cid = lax.axis_index("c")
    sid = lax.axis_index("s")
    base_node = cid * HALF

    row0 = sid * ROWS_PER_TILE
    @pl.when(sid < NS - 1)
    def _():
        pltpu.sync_copy(zeros_hbm.at[pl.ds(row0, ROWS_PER_TILE)],
                        acc.at[pl.ds(row0, ROWS_PER_TILE)])
    @pl.when(sid == NS - 1)
    def _():
        pltpu.sync_copy(zeros_hbm.at[pl.ds(row0, ACC_ROWS - 15 * ROWS_PER_TILE)],
                        acc.at[pl.ds(row0, ACC_ROWS - 15 * ROWS_PER_TILE)])
    plsc.subcore_barrier()

    lane = lax.iota(jnp.int32, 16)
    trash = HALF + (lane + sid) % 24

    eoff = sid * EDGES_PER_TILE

    def _do_chunk(off, count):
        pltpu.sync_copy(src_hbm.at[pl.ds(off, count)], sidx.at[pl.ds(0, count)])
        pltpu.sync_copy(dst_hbm.at[pl.ds(off, count)], dtmp.at[pl.ds(0, count)])
        for k in range(CHUNK // 16):
            if k * 16 < count:
                v = dtmp[pl.ds(k * 16, 16)]
                lv = v - base_node
                oob = (lv < 0) | (lv >= HALF)
                didx[pl.ds(k * 16, 16)] = jnp.where(oob, trash, lv)
            else:
                didx[pl.ds(k * 16, 16)] = trash
                sidx[pl.ds(k * 16, 16)] = jnp.zeros((16,), jnp.int32)
        pltpu.async_copy(x_hbm.at[sidx], rows, gsem).wait()
        pltpu.sync_copy(rows, acc.at[didx], add=True)

    def _chunk_loop(j, _):
        _do_chunk(eoff + j * CHUNK, CHUNK)
        return 0
    lax.fori_loop(0, NFULL, _chunk_loop, 0)
    _do_chunk(eoff + NFULL * CHUNK, TAIL)

    plsc.subcore_barrier()

    @pl.when(sid < NS - 1)
    def _():
        pltpu.sync_copy(acc.at[pl.ds(row0, ROWS_PER_TILE)],
                        out_hbm.at[pl.ds(base_node + row0, ROWS_PER_TILE)])
    @pl.when(sid == NS - 1)
    def _():
        pltpu.sync_copy(acc.at[pl.ds(row0, HALF - 15 * ROWS_PER_TILE)],
                        out_hbm.at[pl.ds(base_node + row0,
                                         HALF - 15 * ROWS_PER_TILE)])


_segsum_sc = functools.partial(
    pl.kernel,
    out_type=jax.ShapeDtypeStruct((NPAD, DP), jnp.float32),
    mesh=_MESH,
    scratch_types=[
        pltpu.VMEM((CHUNK,), jnp.int32),
        pltpu.VMEM((CHUNK,), jnp.int32),
        pltpu.VMEM((CHUNK,), jnp.int32),
        pltpu.VMEM((CHUNK, DP), jnp.float32),
        pltpu.VMEM_SHARED((ACC_ROWS, DP), jnp.float32),
        pltpu.SemaphoreType.DMA,
    ],
    compiler_params=pltpu.CompilerParams(use_tc_tiling_on_sc=False),
)(_segsum_body)


def _segmax_body(h_hbm, b_hbm, out_hbm, rowbuf, bbuf, acc):
    cid = lax.axis_index("c")
    sid = lax.axis_index("s")
    wid = sid * NC + cid
    base = wid * ROWS_PER_TILE

    def _zero_acc(r, _):
        for j in range(HP // 16):
            acc[r, pl.ds(j * 16, 16)] = jnp.zeros((16,), jnp.float32)
        return 0
    lax.fori_loop(0, GM_ROWS, _zero_acc, 0)

    RB = 32
    def _chunk(i, _):
        pltpu.sync_copy(h_hbm.at[pl.ds(base + i * RB, RB)], rowbuf)
        pltpu.sync_copy(b_hbm.at[pl.ds(base + i * RB, RB)], bbuf)
        bvecs = [bbuf[pl.ds(k * 16, 16)] for k in range(RB // 16)]
        for r in range(RB):
            b = bvecs[r // 16][r % 16]
            for j in range(HP // 16):
                sl = pl.ds(j * 16, 16)
                acc[b, sl] = jnp.maximum(acc[b, sl], rowbuf[r, sl])
        return 0
    lax.fori_loop(0, ROWS_PER_TILE // RB, _chunk, 0)

    pltpu.sync_copy(acc.at[pl.ds(0, G)], out_hbm.at[wid])


_segmax_sc = functools.partial(
    pl.kernel,
    out_type=jax.ShapeDtypeStruct((NC * NS, G, HP), jnp.float32),
    mesh=_MESH,
    scratch_types=[
        pltpu.VMEM((32, HP), jnp.float32),
        pltpu.VMEM((32,), jnp.int32),
        pltpu.VMEM((GM_ROWS, HP), jnp.float32),
    ],
    compiler_params=pltpu.CompilerParams(use_tc_tiling_on_sc=False),
)(_segmax_body)


def _mm(a, b):
    return lax.dot_general(a, b, (((1,), (0,)), ((), ())),
                           preferred_element_type=jnp.float32)


def _gin_mlp_block(x_ref, a_ref, w1_ref, b1_ref, w2_ref, b2_ref, o_ref):
    t = (1.0 + EPS) * x_ref[...] + a_ref[...]
    h1 = jnp.maximum(_mm(t, w1_ref[...]) + b1_ref[...], 0.0)
    o_ref[...] = jnp.maximum(_mm(h1, w2_ref[...]) + b2_ref[...], 0.0)


def _gin_mlp(x_pad, agg_pad, w1p, b1, w2p, b2p):
    BLK = 1568
    return pl.pallas_call(
        _gin_mlp_block,
        grid=(NPAD // BLK,),
        in_specs=[
            pl.BlockSpec((BLK, DP), lambda i: (i, 0)),
            pl.BlockSpec((BLK, DP), lambda i: (i, 0)),
            pl.BlockSpec((DP, H), lambda i: (0, 0)),
            pl.BlockSpec((1, H), lambda i: (0, 0)),
            pl.BlockSpec((H, HP), lambda i: (0, 0)),
            pl.BlockSpec((1, HP), lambda i: (0, 0)),
        ],
        out_specs=pl.BlockSpec((BLK, HP), lambda i: (i, 0)),
        out_shape=jax.ShapeDtypeStruct((NPAD, HP), jnp.float32),
    )(x_pad, agg_pad, w1p, b1.reshape(1, H), w2p, b2p.reshape(1, HP))


def _hist_block(t_ref, wr_ref, s_ref):
    t_rep = jnp.broadcast_to(t_ref[...][:, None, :], (8, 32, SEQ)).reshape(256, SEQ)
    iv = lax.broadcasted_iota(jnp.int32, (8, 32, SEQ), 1).reshape(256, SEQ)
    mask = (t_rep == iv).astype(jnp.float32)
    s_ref[...] = _mm(mask, wr_ref[...]).reshape(8, 32, 128)


def _hist_s(target, wr):
    return pl.pallas_call(
        _hist_block,
        grid=(G // 8,),
        in_specs=[
            pl.BlockSpec((8, SEQ), lambda i: (i, 0)),
            pl.BlockSpec((SEQ, 128), lambda i: (0, 0)),
        ],
        out_specs=pl.BlockSpec((8, 32, 128), lambda i: (i, 0, 0)),
        out_shape=jax.ShapeDtypeStruct((G, 32, 128), jnp.float32),
    )(target, wr)


def _convmm_block(a_ref, e2_ref, o_ref):
    o_ref[...] = _mm(a_ref[...], e2_ref[...])


def _conv_mm(a, e2p):
    return pl.pallas_call(
        _convmm_block,
        in_specs=[pl.BlockSpec((G * CH, 1024), lambda: (0, 0)),
                  pl.BlockSpec((1024, 97), lambda: (0, 0))],
        out_specs=pl.BlockSpec((G * CH, 97), lambda: (0, 0)),
        out_shape=jax.ShapeDtypeStruct((G * CH, 97), jnp.float32),
        grid=(),
    )(a, e2p)


def _head_block(gm_ref, fcgw_ref, fcgb_ref, cv_ref, fxw_ref, fxb_ref,
                f1a_ref, f1b_ref, f1bias_ref, f2w_ref, f2b_ref, ow_ref, ob_ref,
                o_ref):
    gm = jnp.max(gm_ref[...], axis=0)
    g = jnp.maximum(_mm(gm, fcgw_ref[...]) + fcgb_ref[...], 0.0)
    xt = _mm(cv_ref[...], fxw_ref[...]) + fxb_ref[...]
    h = jnp.maximum(_mm(g, f1a_ref[...]) + _mm(xt, f1b_ref[...])
                    + f1bias_ref[...], 0.0)
    h2 = jnp.maximum(_mm(h, f2w_ref[...]) + f2b_ref[...], 0.0)
    o_ref[...] = _mm(h2, ow_ref[...]) + ob_ref[...]


def _head(gmax2, fcg_wp, fcg_b, convr, fcxt_wp, fcxt_b_eff, fc1a, fc1b, fc1_b,
          fc2_w, fc2_b, out_wp, out_bp):
    return pl.pallas_call(
        _head_block,
        in_specs=[pl.BlockSpec(s, (lambda s=s: (0,) * len(s))) for s in [
            (NC * NS, G, HP), (HP, OUT), (1, OUT), (G, 392), (392, OUT), (1, OUT),
            (OUT, 1024), (OUT, 1024), (1, 1024), (1024, 512), (1, 512), (512, 8),
            (1, 8),
        ]],
        out_specs=pl.BlockSpec((G, 8), lambda: (0, 0)),
        out_shape=jax.ShapeDtypeStruct((G, 8), jnp.float32),
        grid=(),
    )(gmax2, fcg_wp, fcg_b.reshape(1, OUT), convr, fcxt_wp,
      fcxt_b_eff.reshape(1, OUT), fc1a, fc1b, fc1_b.reshape(1, 1024), fc2_w,
      fc2_b.reshape(1, 512), out_wp, out_bp.reshape(1, 8))


def kernel(x, edge_index, batch, target, gin_w1, gin_b1, gin_w2, gin_b2,
           fcg_w, fcg_b, emb_table, conv_w, conv_b, fcxt_w, fcxt_b,
           fc1_w, fc1_b, fc2_w, fc2_b, out_w, out_b):
    x_pad = jnp.pad(x, ((0, NPAD - N), (0, DP - D)))
    w1p = jnp.pad(gin_w1, ((0, DP - D), (0, 0)))
    w2p = jnp.pad(gin_w2, ((0, 0), (0, HP - H)))
    b2p = jnp.pad(gin_b2, (0, HP - H))
    fcg_wp = jnp.pad(fcg_w, ((0, HP - H), (0, 0)))
    wr = conv_w.transpose(1, 0, 2).reshape(SEQ, CH * KW)
    emb_pad = jnp.pad(emb_table, ((0, 32 - VOC), (0, 0)))
    e2p = jnp.stack([emb_pad[:, k:k + 97] for k in range(KW)], axis=1)
    e2p = e2p.reshape(1024, 97)
    bias_vec = jnp.repeat(conv_b, 97)
    fcxt_b_eff = bias_vec @ fcxt_w + fcxt_b
    fcxt_wp = jnp.pad(fcxt_w, ((0, 4), (0, 0)))
    fc1a = fc1_w[:OUT]
    fc1b = fc1_w[OUT:]
    out_wp = jnp.pad(out_w, ((0, 0), (0, 7)))
    out_bp = jnp.pad(out_b, (0, 7))
    src = edge_index[0]
    dst = edge_index[1]
    batch_pad = jnp.pad(batch, (0, NPAD - N), constant_values=G)

    zeros_acc = jnp.zeros((ACC_ROWS, DP), jnp.float32)
    agg = _segsum_sc(x_pad, src, dst, zeros_acc)
    h2 = _gin_mlp(x_pad, agg, w1p, gin_b1, w2p, b2p)
    gmax2 = _segmax_sc(h2, batch_pad)

    s = _hist_s(target, wr)
    a = s.reshape(G, 32, CH, KW).transpose(0, 2, 1, 3).reshape(G * CH, 32 * KW)
    convf = _conv_mm(a, e2p)
    convr = jnp.pad(convf.reshape(G, XT_IN), ((0, 0), (0, 4)))

    o = _head(gmax2, fcg_wp, fcg_b, convr, fcxt_wp, fcxt_b_eff, fc1a, fc1b,
              fc1_b, fc2_w, fc2_b, out_wp, out_bp)
    return o[:, :1]


# speedup vs baseline: 4.1443x; 1.4074x over previous
import functools

import jax
import jax.numpy as jnp
from jax import lax
from jax.experimental import pallas as pl
from jax.experimental.pallas import tpu as pltpu
from jax.experimental.pallas import tpu_sc as plsc

N = 50000
E = 800000
G = 256
D = 78
H = 312
OUT = 128
EMB = 128
VOC = 26
KW = 32
CH = 4
SEQ = 1000
EPS = 1.583
XT_IN = CH * (EMB - KW + 1)

NPAD = 50176
HP = 320
DP = 80

NC = 2
NS = 16
HALF = 25000
ACC_ROWS = 25024
EDGES_PER_TILE = E // NS
CHUNK = 32
NFULL = EDGES_PER_TILE // CHUNK
TAIL = EDGES_PER_TILE - NFULL * CHUNK

ROWS_PER_TILE = NPAD // (NC * NS)
GM_ROWS = 272

_MESH = plsc.VectorSubcoreMesh(core_axis_name="c", subcore_axis_name="s")


def _segsum_body(x_hbm, src_hbm, dst_hbm, zeros_hbm, out_hbm,
                 sidx0, sidx1, dtmp0, dtmp1, didx0, didx1, rows0, rows1, acc,
                 ssrc0, ssrc1, sdst0, sdst1, gsem0, gsem1, ssem0, ssem1):
    cid = lax.axis_index("c")
    sid = lax.axis_index("s")
    base_node = cid * HALF

    row0 = sid * ROWS_PER_TILE
    @pl.when(sid < NS - 1)
    def _():
        pltpu.sync_copy(zeros_hbm.at[pl.ds(row0, ROWS_PER_TILE)],
                        acc.at[pl.ds(row0, ROWS_PER_TILE)])
    @pl.when(sid == NS - 1)
    def _():
        pltpu.sync_copy(zeros_hbm.at[pl.ds(row0, ACC_ROWS - 15 * ROWS_PER_TILE)],
                        acc.at[pl.ds(row0, ACC_ROWS - 15 * ROWS_PER_TILE)])
    plsc.subcore_barrier()

    lane = lax.iota(jnp.int32, 16)
    trash = HALF + (lane + sid) % 24

    eoff = sid * EDGES_PER_TILE
    sidx = [sidx0, sidx1]
    dtmp = [dtmp0, dtmp1]
    didx = [didx0, didx1]
    rows = [rows0, rows1]
    ssrc = [ssrc0, ssrc1]
    sdst = [sdst0, sdst1]
    gsem = [gsem0, gsem1]
    ssem = [ssem0, ssem1]

    def _prefetch(off, s, count):
        pltpu.async_copy(src_hbm.at[pl.ds(off, count)],
                         sidx[s].at[pl.ds(0, count)], ssrc[s])
        pltpu.async_copy(dst_hbm.at[pl.ds(off, count)],
                         dtmp[s].at[pl.ds(0, count)], sdst[s])

    def _wait_prefetch(off, s, count):
        pltpu.make_async_copy(src_hbm.at[pl.ds(off, count)],
                              sidx[s].at[pl.ds(0, count)], ssrc[s]).wait()
        pltpu.make_async_copy(dst_hbm.at[pl.ds(off, count)],
                              dtmp[s].at[pl.ds(0, count)], sdst[s]).wait()

    def _compute_didx(s, count):
        for k in range(CHUNK // 16):
            if k * 16 < count:
                v = dtmp[s][pl.ds(k * 16, 16)]
                lv = v - base_node
                oob = (lv < 0) | (lv >= HALF)
                didx[s][pl.ds(k * 16, 16)] = jnp.where(oob, trash, lv)
            else:
                didx[s][pl.ds(k * 16, 16)] = trash
                sidx[s][pl.ds(k * 16, 16)] = jnp.zeros((16,), jnp.int32)

    def _start_gather(s):
        pltpu.async_copy(x_hbm.at[sidx[s]], rows[s], gsem[s])

    def _wait_gather(s):
        pltpu.make_async_copy(x_hbm.at[sidx[s]], rows[s], gsem[s]).wait()

    def _start_scatter(s):
        pltpu.async_copy(rows[s], acc.at[didx[s]], ssem[s], add=True)

    def _wait_scatter(s):
        pltpu.make_async_copy(rows[s], acc.at[didx[s]], ssem[s]).wait()

    def _step(off, s, count, next_off, next_count):
        _wait_scatter(s)
        _wait_prefetch(off, s, count)
        _compute_didx(s, count)
        _start_gather(s)
        _wait_gather(1 - s)
        _start_scatter(1 - s)
        if next_count:
            _prefetch(next_off, 1 - s, next_count)

    _prefetch(eoff, 0, CHUNK)
    _wait_prefetch(eoff, 0, CHUNK)
    _compute_didx(0, CHUNK)
    _start_gather(0)
    _prefetch(eoff + CHUNK, 1, CHUNK)
    _wait_prefetch(eoff + CHUNK, 1, CHUNK)
    _compute_didx(1, CHUNK)
    _start_gather(1)
    _wait_gather(0)
    _start_scatter(0)
    _prefetch(eoff + 2 * CHUNK, 0, CHUNK)

    def _pair(t, _):
        off = eoff + (2 + 2 * t) * CHUNK
        _step(off, 0, CHUNK, off + CHUNK, CHUNK)
        _step(off + CHUNK, 1, CHUNK, off + 2 * CHUNK, CHUNK)
        return 0
    lax.fori_loop(0, (NFULL - 4) // 2, _pair, 0)

    off_f = eoff + (NFULL - 2) * CHUNK
    _step(off_f, 0, CHUNK, off_f + CHUNK, CHUNK)
    _step(off_f + CHUNK, 1, CHUNK, off_f + 2 * CHUNK, TAIL)
    _step(off_f + 2 * CHUNK, 0, TAIL, 0, 0)
    _wait_gather(0)
    _start_scatter(0)
    _wait_scatter(1)
    _wait_scatter(0)

    plsc.subcore_barrier()

    @pl.when(sid < NS - 1)
    def _():
        pltpu.sync_copy(acc.at[pl.ds(row0, ROWS_PER_TILE)],
                        out_hbm.at[pl.ds(base_node + row0, ROWS_PER_TILE)])
    @pl.when(sid == NS - 1)
    def _():
        pltpu.sync_copy(acc.at[pl.ds(row0, HALF - 15 * ROWS_PER_TILE)],
                        out_hbm.at[pl.ds(base_node + row0,
                                         HALF - 15 * ROWS_PER_TILE)])


_segsum_sc = functools.partial(
    pl.kernel,
    out_type=jax.ShapeDtypeStruct((NPAD, DP), jnp.float32),
    mesh=_MESH,
    scratch_types=(
        [pltpu.VMEM((CHUNK,), jnp.int32)] * 6
        + [pltpu.VMEM((CHUNK, DP), jnp.float32)] * 2
        + [pltpu.VMEM_SHARED((ACC_ROWS, DP), jnp.float32)]
        + [pltpu.SemaphoreType.DMA] * 8
    ),
    compiler_params=pltpu.CompilerParams(use_tc_tiling_on_sc=False),
)(_segsum_body)


def _segmax_body(h_hbm, b_hbm, out_hbm, rowbuf, bbuf, acc):
    cid = lax.axis_index("c")
    sid = lax.axis_index("s")
    wid = sid * NC + cid
    base = wid * ROWS_PER_TILE

    def _zero_acc(r, _):
        for j in range(HP // 16):
            acc[r, pl.ds(j * 16, 16)] = jnp.zeros((16,), jnp.float32)
        return 0
    lax.fori_loop(0, GM_ROWS, _zero_acc, 0)

    RB = 32
    def _chunk(i, _):
        pltpu.sync_copy(h_hbm.at[pl.ds(base + i * RB, RB)], rowbuf)
        pltpu.sync_copy(b_hbm.at[pl.ds(base + i * RB, RB)], bbuf)
        bvecs = [bbuf[pl.ds(k * 16, 16)] for k in range(RB // 16)]
        for r in range(RB):
            b = bvecs[r // 16][r % 16]
            for j in range(HP // 16):
                sl = pl.ds(j * 16, 16)
                acc[b, sl] = jnp.maximum(acc[b, sl], rowbuf[r, sl])
        return 0
    lax.fori_loop(0, ROWS_PER_TILE // RB, _chunk, 0)

    pltpu.sync_copy(acc.at[pl.ds(0, G)], out_hbm.at[wid])


_segmax_sc = functools.partial(
    pl.kernel,
    out_type=jax.ShapeDtypeStruct((NC * NS, G, HP), jnp.float32),
    mesh=_MESH,
    scratch_types=[
        pltpu.VMEM((32, HP), jnp.float32),
        pltpu.VMEM((32,), jnp.int32),
        pltpu.VMEM((GM_ROWS, HP), jnp.float32),
    ],
    compiler_params=pltpu.CompilerParams(use_tc_tiling_on_sc=False),
)(_segmax_body)


def _mm(a, b):
    return lax.dot_general(a, b, (((1,), (0,)), ((), ())),
                           preferred_element_type=jnp.float32)


def _gin_mlp_block(x_ref, a_ref, w1_ref, b1_ref, w2_ref, b2_ref, o_ref):
    t = (1.0 + EPS) * x_ref[...] + a_ref[...]
    h1 = jnp.maximum(_mm(t, w1_ref[...]) + b1_ref[...], 0.0)
    o_ref[...] = jnp.maximum(_mm(h1, w2_ref[...]) + b2_ref[...], 0.0)


def _gin_mlp(x_pad, agg_pad, w1p, b1, w2p, b2p):
    BLK = 1568
    return pl.pallas_call(
        _gin_mlp_block,
        grid=(NPAD // BLK,),
        in_specs=[
            pl.BlockSpec((BLK, DP), lambda i: (i, 0)),
            pl.BlockSpec((BLK, DP), lambda i: (i, 0)),
            pl.BlockSpec((DP, H), lambda i: (0, 0)),
            pl.BlockSpec((1, H), lambda i: (0, 0)),
            pl.BlockSpec((H, HP), lambda i: (0, 0)),
            pl.BlockSpec((1, HP), lambda i: (0, 0)),
        ],
        out_specs=pl.BlockSpec((BLK, HP), lambda i: (i, 0)),
        out_shape=jax.ShapeDtypeStruct((NPAD, HP), jnp.float32),
    )(x_pad, agg_pad, w1p, b1.reshape(1, H), w2p, b2p.reshape(1, HP))


def _hist_block(t_ref, wr_ref, s_ref):
    t_rep = jnp.broadcast_to(t_ref[...][:, None, :], (8, 32, SEQ)).reshape(256, SEQ)
    iv = lax.broadcasted_iota(jnp.int32, (8, 32, SEQ), 1).reshape(256, SEQ)
    mask = (t_rep == iv).astype(jnp.float32)
    s_ref[...] = _mm(mask, wr_ref[...]).reshape(8, 32, 128)


def _hist_s(target, wr):
    return pl.pallas_call(
        _hist_block,
        grid=(G // 8,),
        in_specs=[
            pl.BlockSpec((8, SEQ), lambda i: (i, 0)),
            pl.BlockSpec((SEQ, 128), lambda i: (0, 0)),
        ],
        out_specs=pl.BlockSpec((8, 32, 128), lambda i: (i, 0, 0)),
        out_shape=jax.ShapeDtypeStruct((G, 32, 128), jnp.float32),
    )(target, wr)


def _convmm_block(a_ref, e2_ref, o_ref):
    o_ref[...] = _mm(a_ref[...], e2_ref[...])


def _conv_mm(a, e2p):
    return pl.pallas_call(
        _convmm_block,
        in_specs=[pl.BlockSpec((G * CH, 1024), lambda: (0, 0)),
                  pl.BlockSpec((1024, 97), lambda: (0, 0))],
        out_specs=pl.BlockSpec((G * CH, 97), lambda: (0, 0)),
        out_shape=jax.ShapeDtypeStruct((G * CH, 97), jnp.float32),
        grid=(),
    )(a, e2p)


def _head_block(gm_ref, fcgw_ref, fcgb_ref, cv_ref, fxw_ref, fxb_ref,
                f1a_ref, f1b_ref, f1bias_ref, f2w_ref, f2b_ref, ow_ref, ob_ref,
                o_ref):
    gm = jnp.max(gm_ref[...], axis=0)
    g = jnp.maximum(_mm(gm, fcgw_ref[...]) + fcgb_ref[...], 0.0)
    xt = _mm(cv_ref[...], fxw_ref[...]) + fxb_ref[...]
    h = jnp.maximum(_mm(g, f1a_ref[...]) + _mm(xt, f1b_ref[...])
                    + f1bias_ref[...], 0.0)
    h2 = jnp.maximum(_mm(h, f2w_ref[...]) + f2b_ref[...], 0.0)
    o_ref[...] = _mm(h2, ow_ref[...]) + ob_ref[...]


def _head(gmax2, fcg_wp, fcg_b, convr, fcxt_wp, fcxt_b_eff, fc1a, fc1b, fc1_b,
          fc2_w, fc2_b, out_wp, out_bp):
    return pl.pallas_call(
        _head_block,
        in_specs=[pl.BlockSpec(s, (lambda s=s: (0,) * len(s))) for s in [
            (NC * NS, G, HP), (HP, OUT), (1, OUT), (G, 392), (392, OUT), (1, OUT),
            (OUT, 1024), (OUT, 1024), (1, 1024), (1024, 512), (1, 512), (512, 8),
            (1, 8),
        ]],
        out_specs=pl.BlockSpec((G, 8), lambda: (0, 0)),
        out_shape=jax.ShapeDtypeStruct((G, 8), jnp.float32),
        grid=(),
    )(gmax2, fcg_wp, fcg_b.reshape(1, OUT), convr, fcxt_wp,
      fcxt_b_eff.reshape(1, OUT), fc1a, fc1b, fc1_b.reshape(1, 1024), fc2_w,
      fc2_b.reshape(1, 512), out_wp, out_bp.reshape(1, 8))


def kernel(x, edge_index, batch, target, gin_w1, gin_b1, gin_w2, gin_b2,
           fcg_w, fcg_b, emb_table, conv_w, conv_b, fcxt_w, fcxt_b,
           fc1_w, fc1_b, fc2_w, fc2_b, out_w, out_b):
    x_pad = jnp.pad(x, ((0, NPAD - N), (0, DP - D)))
    w1p = jnp.pad(gin_w1, ((0, DP - D), (0, 0)))
    w2p = jnp.pad(gin_w2, ((0, 0), (0, HP - H)))
    b2p = jnp.pad(gin_b2, (0, HP - H))
    fcg_wp = jnp.pad(fcg_w, ((0, HP - H), (0, 0)))
    wr = conv_w.transpose(1, 0, 2).reshape(SEQ, CH * KW)
    emb_pad = jnp.pad(emb_table, ((0, 32 - VOC), (0, 0)))
    e2p = jnp.stack([emb_pad[:, k:k + 97] for k in range(KW)], axis=1)
    e2p = e2p.reshape(1024, 97)
    bias_vec = jnp.repeat(conv_b, 97)
    fcxt_b_eff = bias_vec @ fcxt_w + fcxt_b
    fcxt_wp = jnp.pad(fcxt_w, ((0, 4), (0, 0)))
    fc1a = fc1_w[:OUT]
    fc1b = fc1_w[OUT:]
    out_wp = jnp.pad(out_w, ((0, 0), (0, 7)))
    out_bp = jnp.pad(out_b, (0, 7))
    src = edge_index[0]
    dst = edge_index[1]
    batch_pad = jnp.pad(batch, (0, NPAD - N), constant_values=G)

    zeros_acc = jnp.zeros((ACC_ROWS, DP), jnp.float32)
    agg = _segsum_sc(x_pad, src, dst, zeros_acc)
    h2 = _gin_mlp(x_pad, agg, w1p, gin_b1, w2p, b2p)
    gmax2 = _segmax_sc(h2, batch_pad)

    s = _hist_s(target, wr)
    a = s.reshape(G, 32, CH, KW).transpose(0, 2, 1, 3).reshape(G * CH, 32 * KW)
    convf = _conv_mm(a, e2p)
    convr = jnp.pad(convf.reshape(G, XT_IN), ((0, 0), (0, 4)))

    o = _head(gmax2, fcg_wp, fcg_b, convr, fcxt_wp, fcxt_b_eff, fc1a, fc1b,
              fc1_b, fc2_w, fc2_b, out_wp, out_bp)
    return o[:, :1]
